# baseline (speedup 1.0000x reference)
import jax
import jax.numpy as jnp
from jax.experimental import pallas as pl
from jax.experimental.pallas import tpu as pltpu


def _top2_combine(logits):
    E = logits.shape[-1]
    eids = jax.lax.broadcasted_iota(jnp.int32, logits.shape, 1)
    m1 = jnp.max(logits, axis=1, keepdims=True)
    i1 = jnp.min(jnp.where(logits == m1, eids, E), axis=1, keepdims=True)
    mask1 = eids == i1
    masked = jnp.where(mask1, -jnp.inf, logits)
    m2 = jnp.max(masked, axis=1, keepdims=True)
    i2 = jnp.min(jnp.where(masked == m2, eids, E), axis=1, keepdims=True)
    mask2 = eids == i2
    e2 = jnp.exp(m2 - m1)
    w1 = 1.0 / (1.0 + e2)
    w2 = e2 / (1.0 + e2)
    return w1 * mask1.astype(logits.dtype) + w2 * mask2.astype(logits.dtype)


_CHUNK = 512


def _moe_kernel(x_ref, wg_ref, we_ref, be_ref, out_ref, comb_ref):
    e = pl.program_id(0)
    T = x_ref.shape[0]
    n_chunks = T // _CHUNK

    @pl.when(e == 0)
    def _gate():
        for i in range(n_chunks):
            rows = pl.ds(i * _CHUNK, _CHUNK)
            x = x_ref[rows, :]
            logits = jnp.dot(x, wg_ref[...],
                             preferred_element_type=jnp.float32)
            combine = _top2_combine(logits)
            comb_ref[rows, :] = combine
            out_ref[rows, :] = jnp.dot(combine, be_ref[...],
                                       preferred_element_type=jnp.float32)

    web = we_ref[0].astype(jnp.bfloat16)
    for i in range(n_chunks):
        rows = pl.ds(i * _CHUNK, _CHUNK)
        combine = comb_ref[rows, :]
        eids = jax.lax.broadcasted_iota(jnp.int32, combine.shape, 1)
        c = jnp.sum(jnp.where(eids == e, combine, 0.0), axis=1, keepdims=True)
        ye = jnp.dot(x_ref[rows, :].astype(jnp.bfloat16), web,
                     preferred_element_type=jnp.float32)
        out_ref[rows, :] += c * ye


def kernel(inputs, Wg, We, be):
    D = inputs.shape[-1]
    E = We.shape[0]
    xf = inputs.reshape(-1, D)
    T = xf.shape[0]
    out = pl.pallas_call(
        _moe_kernel,
        grid=(E,),
        in_specs=[
            pl.BlockSpec((T, D), lambda e: (0, 0)),
            pl.BlockSpec(Wg.shape, lambda e: (0, 0)),
            pl.BlockSpec((1, D, D), lambda e: (e, 0, 0)),
            pl.BlockSpec(be.shape, lambda e: (0, 0)),
        ],
        out_specs=pl.BlockSpec((T, D), lambda e: (0, 0)),
        out_shape=jax.ShapeDtypeStruct((T, D), inputs.dtype),
        scratch_shapes=[pltpu.VMEM((T, E), jnp.float32)],
        compiler_params=pltpu.CompilerParams(
            dimension_semantics=("arbitrary",)),
    )(xf, Wg, We, be)
    return out.reshape(inputs.shape)
